# baseline (speedup 1.0000x reference)
import functools

import jax
import jax.numpy as jnp
from jax import lax
from jax.experimental import pallas as pl
from jax.experimental.pallas import tpu as pltpu
from jax.experimental.pallas import tpu_sc as plsc

MARGIN = 0.2
_NC, _NS, _LANES = 2, 16, 16
_NSET = 4


def _tc_body(R, NB, n, dis_ref, tcol_ref, trow_ref, emb_ref, embblk_ref,
             out_ref):
    i = pl.program_id(0)
    dis = dis_ref[...]
    tcol = tcol_ref[...]
    trow = trow_ref[...]
    col = lax.broadcasted_iota(jnp.int32, (R, n), 1)
    row = lax.broadcasted_iota(jnp.int32, (R, n), 0) + i * R
    same = tcol == trow
    neg_inf = jnp.float32(-jnp.inf)
    pos_inf = jnp.float32(jnp.inf)
    pval = jnp.where(same & (col != row), dis, neg_inf)
    nval = jnp.where(same, pos_inf, dis)
    pmax = jnp.max(pval, axis=1, keepdims=True)
    nmin = jnp.min(nval, axis=1, keepdims=True)
    big = jnp.int32(n)
    pidx = jnp.min(jnp.where(pval == pmax, col, big), axis=1, keepdims=True)
    nidx = jnp.min(jnp.where(nval == nmin, col, big), axis=1, keepdims=True)
    onehot_p = (col == pidx).astype(jnp.float32)
    onehot_n = (col == nidx).astype(jnp.float32)
    emb = emb_ref[...]
    p = jnp.dot(onehot_p, emb, preferred_element_type=jnp.float32)
    ng = jnp.dot(onehot_n, emb, preferred_element_type=jnp.float32)
    a = embblk_ref[...]
    ap = jnp.sum((a - p) ** 2, axis=1, keepdims=True)
    an = jnp.sum((a - ng) ** 2, axis=1, keepdims=True)
    losses = jnp.maximum(ap - an + jnp.float32(MARGIN), jnp.float32(0.0))
    part = jnp.sum(losses) / jnp.float32(n)

    @pl.when(i == 0)
    def _init():
        out_ref[0, 0] = jnp.float32(0.0)

    out_ref[0, 0] += part


def _tc_partial(embeddings, dis, target, tc_rows, R=256):
    n, d = embeddings.shape
    NB = tc_rows // R
    tcol = target.reshape(n, 1)
    trow = target.reshape(1, n)
    out = pl.pallas_call(
        functools.partial(_tc_body, R, NB, n),
        grid=(NB,),
        in_specs=[
            pl.BlockSpec((R, n), lambda i: (i, 0)),
            pl.BlockSpec((R, 1), lambda i: (i, 0)),
            pl.BlockSpec((1, n), lambda i: (0, 0)),
            pl.BlockSpec((n, d), lambda i: (0, 0)),
            pl.BlockSpec((R, d), lambda i: (i, 0)),
        ],
        out_specs=pl.BlockSpec(memory_space=pltpu.SMEM),
        out_shape=jax.ShapeDtypeStruct((1, 1), jnp.float32),
        compiler_params=pltpu.CompilerParams(
            dimension_semantics=("arbitrary",),
        ),
    )(dis, tcol, trow, embeddings, embeddings)
    return out[0, 0]


def _make_sc_part(n, d, S):
    NW = _NC * _NS
    RP = S // NW
    CH = d // _LANES
    NCHUNK = n // _LANES
    mesh = plsc.VectorSubcoreMesh(core_axis_name="c", subcore_axis_name="s")

    @functools.partial(
        pl.kernel,
        mesh=mesh,
        out_type=jax.ShapeDtypeStruct((_NC, _LANES), jnp.float32),
        scratch_types=[
            pltpu.VMEM((n,), jnp.int32),
            pltpu.VMEM((2, 8, n), jnp.float32),
            pltpu.VMEM((RP,), jnp.int32),
            pltpu.VMEM((RP,), jnp.int32),
            pltpu.VMEM((RP, d), jnp.float32),
            pltpu.VMEM((RP, d), jnp.float32),
            pltpu.VMEM((RP, d), jnp.float32),
            pltpu.VMEM((_NS, _LANES), jnp.float32),
            pltpu.VMEM((_LANES,), jnp.float32),
            pltpu.HBM((_NC, _NS, _LANES), jnp.float32),
            pltpu.SemaphoreType.DMA,
            pltpu.SemaphoreType.DMA,
            pltpu.SemaphoreType.DMA,
            pltpu.SemaphoreType.DMA,
            pltpu.SemaphoreType.DMA,
        ],
        compiler_params=pltpu.CompilerParams(needs_layout_passes=False),
    )
    def sc_part(emb_hbm, dis_hbm, tgt_hbm, out_hbm,
                tgt_v, rows2, pidx_v, nidx_v, arow, prow, nrow,
                buf2, outbuf, stage, semA, semB, sg1, sg2, sg3):
        cid = lax.axis_index("c")
        sid = lax.axis_index("s")
        wid = sid * _NC + cid
        gbase = (n - S) + wid * RP
        lanes = lax.iota(jnp.int32, _LANES)
        neg_inf = jnp.float32(-jnp.inf)
        pos_inf = jnp.float32(jnp.inf)

        pltpu.sync_copy(tgt_hbm, tgt_v)
        pltpu.async_copy(dis_hbm.at[pl.ds(gbase, 8)], rows2.at[0], semA)
        pltpu.async_copy(dis_hbm.at[pl.ds(gbase + 8, 8)], rows2.at[1], semB)
        sems = (semA, semB)
        NG = RP // 8

        def group(g, carry):
            for b in range(2):
                gg = g * 2 + b
                pltpu.make_async_copy(dis_hbm.at[pl.ds(gbase, 8)],
                                      rows2.at[b], sems[b]).wait()
                for rr in range(8):
                    r = gg * 8 + rr
                    gr = gbase + r
                    lab = plsc.load_gather(
                        tgt_v, [jnp.full((_LANES,), gr, jnp.int32)])
                    dbase = (gr // _LANES) * _LANES
                    dchunk = rows2[b, rr, pl.ds(dbase, _LANES)]
                    rows2[b, rr, pl.ds(dbase, _LANES)] = jnp.where(
                        lanes == (gr % _LANES), neg_inf, dchunk)

                    def step(c, carry, rr=rr, lab=lab):
                        out = []
                        for k in range(_NSET):
                            pmax, pidx, nmin, nidx = carry[k]
                            cc = c * _NSET + k
                            d16 = rows2[b, rr, pl.ds(cc * _LANES, _LANES)]
                            t16 = tgt_v[pl.ds(cc * _LANES, _LANES)]
                            col16 = lanes + cc * _LANES
                            same = t16 == lab
                            pu = (d16 > pmax) & same
                            nu = (d16 < nmin) & (~same)
                            pmax = jnp.where(pu, d16, pmax)
                            pidx = jnp.where(pu, col16, pidx)
                            nmin = jnp.where(nu, d16, nmin)
                            nidx = jnp.where(nu, col16, nidx)
                            out.append((pmax, pidx, nmin, nidx))
                        return tuple(out)

                    init1 = (jnp.full((_LANES,), neg_inf, jnp.float32),
                             jnp.zeros((_LANES,), jnp.int32),
                             jnp.full((_LANES,), pos_inf, jnp.float32),
                             jnp.zeros((_LANES,), jnp.int32))
                    sets = lax.fori_loop(0, NCHUNK // _NSET, step,
                                         (init1,) * _NSET, unroll=2)
                    big = jnp.int32(n)
                    pmax = sets[0][0]
                    nmin = sets[0][2]
                    for k in range(1, _NSET):
                        pmax = jnp.maximum(pmax, sets[k][0])
                        nmin = jnp.minimum(nmin, sets[k][2])
                    m = jnp.max(pmax)
                    mi = jnp.min(nmin)
                    pcand = jnp.where(sets[0][0] == m, sets[0][1], big)
                    ncand = jnp.where(sets[0][2] == mi, sets[0][3], big)
                    for k in range(1, _NSET):
                        pcand = jnp.minimum(
                            pcand, jnp.where(sets[k][0] == m, sets[k][1], big))
                        ncand = jnp.minimum(
                            ncand, jnp.where(sets[k][2] == mi, sets[k][3], big))
                    pr = jnp.min(pcand)
                    nr = jnp.min(ncand)
                    ridx = jnp.full((_LANES,), r, jnp.int32)
                    lane0 = lanes == 0
                    plsc.store_scatter(pidx_v, [ridx],
                                       jnp.full((_LANES,), pr, jnp.int32),
                                       mask=lane0)
                    plsc.store_scatter(nidx_v, [ridx],
                                       jnp.full((_LANES,), nr, jnp.int32),
                                       mask=lane0)

                @pl.when(gg + 2 < NG)
                def _prefetch():
                    pltpu.async_copy(
                        dis_hbm.at[pl.ds(gbase + (gg + 2) * 8, 8)],
                        rows2.at[b], sems[b])

            return carry

        lax.fori_loop(0, NG // 2, group, 0)

        cp1 = pltpu.async_copy(emb_hbm.at[pidx_v], prow, sg1)
        cp2 = pltpu.async_copy(emb_hbm.at[nidx_v], nrow, sg2)
        cp3 = pltpu.async_copy(emb_hbm.at[pl.ds(gbase, RP)], arow, sg3)
        cp1.wait()
        cp2.wait()
        cp3.wait()

        def row_body(r, total):
            acc = jnp.zeros((_LANES,), jnp.float32)
            for c in range(CH):
                a = arow[r, pl.ds(c * _LANES, _LANES)]
                p = prow[r, pl.ds(c * _LANES, _LANES)]
                ng = nrow[r, pl.ds(c * _LANES, _LANES)]
                dp = a - p
                dn = a - ng
                acc = acc + dp * dp - dn * dn
            t = jnp.sum(acc)
            return total + jnp.maximum(t + jnp.float32(MARGIN),
                                       jnp.float32(0.0))

        total = lax.fori_loop(0, RP, row_body, jnp.float32(0.0))
        total = total * jnp.float32(1.0 / n)

        zeros16 = jnp.zeros((_LANES,), jnp.float32)
        outbuf[...] = jnp.where(lanes == 0,
                                jnp.full((_LANES,), total, jnp.float32),
                                zeros16)
        pltpu.sync_copy(outbuf, stage.at[cid, sid])
        plsc.subcore_barrier()

        @pl.when(sid == 0)
        def _reduce():
            pltpu.sync_copy(stage.at[cid], buf2)
            acc = jnp.zeros((_LANES,), jnp.float32)
            for r2 in range(_NS):
                acc = acc + buf2[r2, :]
            tot = jnp.sum(acc)
            outbuf[...] = jnp.where(lanes == 0,
                                    jnp.full((_LANES,), tot, jnp.float32),
                                    zeros16)
            pltpu.sync_copy(outbuf, out_hbm.at[cid])

    return sc_part


def kernel(embeddings, dis, target):
    n, d = embeddings.shape
    S = 512
    sc_part = _make_sc_part(n, d, S)
    sc_out = sc_part(embeddings, dis, target)
    tc_part = _tc_partial(embeddings, dis, target, n - S)
    return sc_out[0, 0] + sc_out[1, 0] + tc_part


# speedup vs baseline: 1.1029x; 1.1029x over previous
import functools

import jax
import jax.numpy as jnp
from jax import lax
from jax.experimental import pallas as pl
from jax.experimental.pallas import tpu as pltpu
from jax.experimental.pallas import tpu_sc as plsc

MARGIN = 0.2
_NC, _NS, _LANES = 2, 16, 16
_NSET = 4


def _tc_body(R, NB, n, dis_ref, tcol_ref, trow_ref, emb_ref, embblk_ref,
             out_ref):
    i = pl.program_id(0)
    dis = dis_ref[...]
    tcol = tcol_ref[...]
    trow = trow_ref[...]
    col = lax.broadcasted_iota(jnp.int32, (R, n), 1)
    row = lax.broadcasted_iota(jnp.int32, (R, n), 0) + i * R
    same = tcol == trow
    neg_inf = jnp.float32(-jnp.inf)
    pos_inf = jnp.float32(jnp.inf)
    pval = jnp.where(same & (col != row), dis, neg_inf)
    nval = jnp.where(same, pos_inf, dis)
    pmax = jnp.max(pval, axis=1, keepdims=True)
    nmin = jnp.min(nval, axis=1, keepdims=True)
    big = jnp.int32(n)
    pidx = jnp.min(jnp.where(pval == pmax, col, big), axis=1, keepdims=True)
    nidx = jnp.min(jnp.where(nval == nmin, col, big), axis=1, keepdims=True)
    onehot_p = (col == pidx).astype(jnp.float32)
    onehot_n = (col == nidx).astype(jnp.float32)
    emb = emb_ref[...]
    p = jnp.dot(onehot_p, emb, preferred_element_type=jnp.float32)
    ng = jnp.dot(onehot_n, emb, preferred_element_type=jnp.float32)
    a = embblk_ref[...]
    ap = jnp.sum((a - p) ** 2, axis=1, keepdims=True)
    an = jnp.sum((a - ng) ** 2, axis=1, keepdims=True)
    losses = jnp.maximum(ap - an + jnp.float32(MARGIN), jnp.float32(0.0))
    part = jnp.sum(losses) / jnp.float32(n)

    @pl.when(i == 0)
    def _init():
        out_ref[0, 0] = jnp.float32(0.0)

    out_ref[0, 0] += part


def _tc_partial(embeddings, dis, target, tc_rows, R=256):
    n, d = embeddings.shape
    NB = tc_rows // R
    tcol = target.reshape(n, 1)
    trow = target.reshape(1, n)
    out = pl.pallas_call(
        functools.partial(_tc_body, R, NB, n),
        grid=(NB,),
        in_specs=[
            pl.BlockSpec((R, n), lambda i: (i, 0)),
            pl.BlockSpec((R, 1), lambda i: (i, 0)),
            pl.BlockSpec((1, n), lambda i: (0, 0)),
            pl.BlockSpec((n, d), lambda i: (0, 0)),
            pl.BlockSpec((R, d), lambda i: (i, 0)),
        ],
        out_specs=pl.BlockSpec(memory_space=pltpu.SMEM),
        out_shape=jax.ShapeDtypeStruct((1, 1), jnp.float32),
        compiler_params=pltpu.CompilerParams(
            dimension_semantics=("arbitrary",),
        ),
    )(dis, tcol, trow, embeddings, embeddings)
    return out[0, 0]


def _make_sc_part(n, d, S):
    NW = _NC * _NS
    RP = S // NW
    CH = d // _LANES
    NCHUNK = n // _LANES
    mesh = plsc.VectorSubcoreMesh(core_axis_name="c", subcore_axis_name="s")

    @functools.partial(
        pl.kernel,
        mesh=mesh,
        out_type=jax.ShapeDtypeStruct((_NC, _LANES), jnp.float32),
        scratch_types=[
            pltpu.VMEM((n,), jnp.int32),
            pltpu.VMEM((2, 8, n), jnp.float32),
            pltpu.VMEM((RP,), jnp.int32),
            pltpu.VMEM((RP,), jnp.int32),
            pltpu.VMEM((RP, d), jnp.float32),
            pltpu.VMEM((RP, d), jnp.float32),
            pltpu.VMEM((RP, d), jnp.float32),
            pltpu.VMEM((_NS, _LANES), jnp.float32),
            pltpu.VMEM((_LANES,), jnp.float32),
            pltpu.HBM((_NC, _NS, _LANES), jnp.float32),
            pltpu.SemaphoreType.DMA,
            pltpu.SemaphoreType.DMA,
            pltpu.SemaphoreType.DMA,
            pltpu.SemaphoreType.DMA,
            pltpu.SemaphoreType.DMA,
        ],
        compiler_params=pltpu.CompilerParams(needs_layout_passes=False),
    )
    def sc_part(emb_hbm, dis_hbm, tgt_hbm, out_hbm,
                tgt_v, rows2, pidx_v, nidx_v, arow, prow, nrow,
                buf2, outbuf, stage, semA, semB, sg1, sg2, sg3):
        cid = lax.axis_index("c")
        sid = lax.axis_index("s")
        wid = sid * _NC + cid
        gbase = (n - S) + wid * RP
        lanes = lax.iota(jnp.int32, _LANES)
        neg_inf = jnp.float32(-jnp.inf)
        pos_inf = jnp.float32(jnp.inf)

        pltpu.sync_copy(tgt_hbm, tgt_v)
        pltpu.async_copy(dis_hbm.at[pl.ds(gbase, 8)], rows2.at[0], semA)
        pltpu.async_copy(dis_hbm.at[pl.ds(gbase + 8, 8)], rows2.at[1], semB)
        sems = (semA, semB)
        NG = RP // 8

        def group(g, carry):
            for b in range(2):
                gg = g * 2 + b
                pltpu.make_async_copy(dis_hbm.at[pl.ds(gbase, 8)],
                                      rows2.at[b], sems[b]).wait()
                for rr in range(8):
                    r = gg * 8 + rr
                    gr = gbase + r
                    lab = plsc.load_gather(
                        tgt_v, [jnp.full((_LANES,), gr, jnp.int32)])
                    dbase = (gr // _LANES) * _LANES
                    dchunk = rows2[b, rr, pl.ds(dbase, _LANES)]
                    rows2[b, rr, pl.ds(dbase, _LANES)] = jnp.where(
                        lanes == (gr % _LANES), neg_inf, dchunk)

                    def step(c, carry, rr=rr, lab=lab):
                        out = []
                        for k in range(_NSET):
                            pmax, pidx, nmin, nidx = carry[k]
                            cc = c * _NSET + k
                            d16 = rows2[b, rr, pl.ds(cc * _LANES, _LANES)]
                            t16 = tgt_v[pl.ds(cc * _LANES, _LANES)]
                            col16 = lanes + cc * _LANES
                            same = t16 == lab
                            pu = (d16 > pmax) & same
                            nu = (d16 < nmin) & (~same)
                            pmax = jnp.where(pu, d16, pmax)
                            pidx = jnp.where(pu, col16, pidx)
                            nmin = jnp.where(nu, d16, nmin)
                            nidx = jnp.where(nu, col16, nidx)
                            out.append((pmax, pidx, nmin, nidx))
                        return tuple(out)

                    init1 = (jnp.full((_LANES,), neg_inf, jnp.float32),
                             jnp.zeros((_LANES,), jnp.int32),
                             jnp.full((_LANES,), pos_inf, jnp.float32),
                             jnp.zeros((_LANES,), jnp.int32))
                    sets = plsc.parallel_loop(
                        0, NCHUNK // _NSET, 1, unroll=2,
                        carry=(init1,) * _NSET)(step)
                    big = jnp.int32(n)
                    pmax = sets[0][0]
                    nmin = sets[0][2]
                    for k in range(1, _NSET):
                        pmax = jnp.maximum(pmax, sets[k][0])
                        nmin = jnp.minimum(nmin, sets[k][2])
                    m = jnp.max(pmax)
                    mi = jnp.min(nmin)
                    pcand = jnp.where(sets[0][0] == m, sets[0][1], big)
                    ncand = jnp.where(sets[0][2] == mi, sets[0][3], big)
                    for k in range(1, _NSET):
                        pcand = jnp.minimum(
                            pcand, jnp.where(sets[k][0] == m, sets[k][1], big))
                        ncand = jnp.minimum(
                            ncand, jnp.where(sets[k][2] == mi, sets[k][3], big))
                    pr = jnp.min(pcand)
                    nr = jnp.min(ncand)
                    ridx = jnp.full((_LANES,), r, jnp.int32)
                    lane0 = lanes == 0
                    plsc.store_scatter(pidx_v, [ridx],
                                       jnp.full((_LANES,), pr, jnp.int32),
                                       mask=lane0)
                    plsc.store_scatter(nidx_v, [ridx],
                                       jnp.full((_LANES,), nr, jnp.int32),
                                       mask=lane0)

                @pl.when(gg + 2 < NG)
                def _prefetch():
                    pltpu.async_copy(
                        dis_hbm.at[pl.ds(gbase + (gg + 2) * 8, 8)],
                        rows2.at[b], sems[b])

            return carry

        lax.fori_loop(0, NG // 2, group, 0)

        cp1 = pltpu.async_copy(emb_hbm.at[pidx_v], prow, sg1)
        cp2 = pltpu.async_copy(emb_hbm.at[nidx_v], nrow, sg2)
        cp3 = pltpu.async_copy(emb_hbm.at[pl.ds(gbase, RP)], arow, sg3)
        cp1.wait()
        cp2.wait()
        cp3.wait()

        def row_body(r, total):
            acc = jnp.zeros((_LANES,), jnp.float32)
            for c in range(CH):
                a = arow[r, pl.ds(c * _LANES, _LANES)]
                p = prow[r, pl.ds(c * _LANES, _LANES)]
                ng = nrow[r, pl.ds(c * _LANES, _LANES)]
                dp = a - p
                dn = a - ng
                acc = acc + dp * dp - dn * dn
            t = jnp.sum(acc)
            return total + jnp.maximum(t + jnp.float32(MARGIN),
                                       jnp.float32(0.0))

        total = lax.fori_loop(0, RP, row_body, jnp.float32(0.0))
        total = total * jnp.float32(1.0 / n)

        zeros16 = jnp.zeros((_LANES,), jnp.float32)
        outbuf[...] = jnp.where(lanes == 0,
                                jnp.full((_LANES,), total, jnp.float32),
                                zeros16)
        pltpu.sync_copy(outbuf, stage.at[cid, sid])
        plsc.subcore_barrier()

        @pl.when(sid == 0)
        def _reduce():
            pltpu.sync_copy(stage.at[cid], buf2)
            acc = jnp.zeros((_LANES,), jnp.float32)
            for r2 in range(_NS):
                acc = acc + buf2[r2, :]
            tot = jnp.sum(acc)
            outbuf[...] = jnp.where(lanes == 0,
                                    jnp.full((_LANES,), tot, jnp.float32),
                                    zeros16)
            pltpu.sync_copy(outbuf, out_hbm.at[cid])

    return sc_part


def kernel(embeddings, dis, target):
    n, d = embeddings.shape
    S = 1024
    sc_part = _make_sc_part(n, d, S)
    sc_out = sc_part(embeddings, dis, target)
    tc_part = _tc_partial(embeddings, dis, target, n - S)
    return sc_out[0, 0] + sc_out[1, 0] + tc_part
